# baseline (speedup 1.0000x reference)
import functools

import jax
import jax.numpy as jnp
from jax import lax
from jax.experimental import pallas as pl
from jax.experimental.pallas import tpu as pltpu
from jax.experimental.pallas import tpu_sc as plsc

_NUM_CLASSES = 1000000
_HIDDEN = 64
_BATCH = 16384

_INFO = plsc.get_sparse_core_info()
_NC = _INFO.num_cores
_NS = _INFO.num_subcores
_L = _INFO.num_lanes
_NW = _NC * _NS
_B_PER_W = _BATCH // _NW

_mesh = plsc.VectorSubcoreMesh(core_axis_name="c", subcore_axis_name="s")


@functools.partial(
    pl.kernel,
    mesh=_mesh,
    out_type=jax.ShapeDtypeStruct((_BATCH * _HIDDEN,), jnp.float32),
    scratch_types=[
        pltpu.SMEM((_B_PER_W,), jnp.int32),
        pltpu.SMEM((_B_PER_W,), jnp.int32),
        pltpu.VMEM_SHARED((_NW, _B_PER_W), jnp.int32),
        pltpu.VMEM_SHARED((_NW, _B_PER_W), jnp.int32),
        pltpu.VMEM((_B_PER_W * _HIDDEN,), jnp.float32),
        pltpu.SemaphoreType.DMA,
    ],
)
def _embed(labels_hbm, drop_hbm, table_hbm, out_hbm,
           lbl_s, drop_s, lbl_sp, drop_sp, rows_v, sem):
    wid = lax.axis_index("s") * _NC + lax.axis_index("c")
    base = wid * _B_PER_W
    pltpu.sync_copy(labels_hbm.at[pl.ds(base, _B_PER_W)], lbl_sp.at[wid])
    pltpu.sync_copy(drop_hbm.at[pl.ds(base, _B_PER_W)], drop_sp.at[wid])
    pltpu.sync_copy(lbl_sp.at[wid], lbl_s)
    pltpu.sync_copy(drop_sp.at[wid], drop_s)

    def body(i, _):
        r = lax.select(drop_s[i] != 0, _NUM_CLASSES, lbl_s[i])
        pltpu.async_copy(
            table_hbm.at[pl.ds(r * _HIDDEN, _HIDDEN)],
            rows_v.at[pl.ds(i * _HIDDEN, _HIDDEN)],
            sem,
        )
        return 0

    lax.fori_loop(0, _B_PER_W, body, 0)
    pltpu.make_async_copy(
        table_hbm.at[pl.ds(0, _B_PER_W * _HIDDEN)], rows_v, sem
    ).wait()
    pltpu.sync_copy(rows_v, out_hbm.at[pl.ds(base * _HIDDEN,
                                             _B_PER_W * _HIDDEN)])


def kernel(labels, force_drop_ids, embedding_table):
    lbl = labels.astype(jnp.int32)
    drop = force_drop_ids.astype(jnp.int32)
    flat = jnp.reshape(embedding_table, ((_NUM_CLASSES + 1) * _HIDDEN,))
    out_flat = _embed(lbl, drop, flat)
    return out_flat.reshape(_BATCH, _HIDDEN)


# speedup vs baseline: 1.7188x; 1.7188x over previous
import functools

import jax
import jax.numpy as jnp
from jax import lax
from jax.experimental import pallas as pl
from jax.experimental.pallas import tpu as pltpu
from jax.experimental.pallas import tpu_sc as plsc

_NUM_CLASSES = 1000000
_HIDDEN = 64
_BATCH = 16384

_INFO = plsc.get_sparse_core_info()
_NC = _INFO.num_cores
_NS = _INFO.num_subcores
_L = _INFO.num_lanes
_NW = _NC * _NS
_B_PER_W = _BATCH // _NW

_mesh = plsc.VectorSubcoreMesh(core_axis_name="c", subcore_axis_name="s")


@functools.partial(
    pl.kernel,
    mesh=_mesh,
    out_type=jax.ShapeDtypeStruct((_BATCH, _HIDDEN), jnp.float32),
    scratch_types=[
        pltpu.SMEM((_B_PER_W,), jnp.int32),
        pltpu.SMEM((_B_PER_W,), jnp.int32),
        pltpu.VMEM_SHARED((_NW, _B_PER_W), jnp.int32),
        pltpu.VMEM_SHARED((_NW, _B_PER_W), jnp.int32),
        pltpu.VMEM((_B_PER_W, _HIDDEN), jnp.float32),
        pltpu.SemaphoreType.DMA,
    ],
    compiler_params=pltpu.CompilerParams(needs_layout_passes=False),
)
def _embed(labels_hbm, drop_hbm, table_hbm, out_hbm,
           lbl_s, drop_s, lbl_sp, drop_sp, rows_v, sem):
    wid = lax.axis_index("s") * _NC + lax.axis_index("c")
    base = wid * _B_PER_W
    pltpu.sync_copy(labels_hbm.at[pl.ds(base, _B_PER_W)], lbl_sp.at[wid])
    pltpu.sync_copy(drop_hbm.at[pl.ds(base, _B_PER_W)], drop_sp.at[wid])
    pltpu.sync_copy(lbl_sp.at[wid], lbl_s)
    pltpu.sync_copy(drop_sp.at[wid], drop_s)

    def body(i, _):
        r = lax.select(drop_s[i] != 0, _NUM_CLASSES, lbl_s[i])
        pltpu.async_copy(
            table_hbm.at[pl.ds(r, 1)],
            rows_v.at[pl.ds(i, 1)],
            sem,
        )
        return 0

    lax.fori_loop(0, _B_PER_W, body, 0)
    pltpu.make_async_copy(table_hbm.at[pl.ds(0, _B_PER_W)], rows_v, sem).wait()
    pltpu.sync_copy(rows_v, out_hbm.at[pl.ds(base, _B_PER_W)])


def kernel(labels, force_drop_ids, embedding_table):
    lbl = labels.astype(jnp.int32)
    drop = force_drop_ids.astype(jnp.int32)
    return _embed(lbl, drop, embedding_table)
